# baseline (speedup 1.0000x reference)
import jax
import jax.numpy as jnp
from jax.experimental import pallas as pl
from jax.experimental.pallas import tpu as pltpu

K = 64
RB = 128
NF = 16
TW = 512
NE = 3
_NEG = -3.0e38


def _body(x_ref, wt_ref, z_ref, s_ref):
    i = pl.program_id(0)
    cur = jax.lax.rem(i, 2)
    prv = 1 - cur

    x = x_ref[...]
    xn = x * jax.lax.rsqrt(jnp.maximum(jnp.sum(x * x, axis=1, keepdims=True),
                                       1e-24))

    hi0 = jnp.full((RB, 1), 1.000001, jnp.float32)
    lo0 = jnp.zeros_like(hi0)

    def fstep(it, carry):
        lo, hi, chi = carry
        sl = pl.ds(it * TW, TW)
        s_ref[cur, :, sl] = jnp.dot(xn, wt_ref[:, sl],
                                    preferred_element_type=jnp.float32)
        mid = (lo + hi) * 0.5
        cnt = jnp.sum((s_ref[prv] >= mid).astype(jnp.float32), axis=1,
                      keepdims=True)
        ge = cnt >= K
        return (jnp.where(ge, mid, lo), jnp.where(ge, hi, mid),
                jnp.where(ge, chi, cnt))

    lo, hi, chi = jax.lax.fori_loop(
        0, NF, fstep, (lo0, hi0, jnp.zeros_like(hi0)))

    sp = s_ref[prv]
    j = K - chi

    cur_v = hi
    t = jnp.full_like(hi, _NEG)
    for e in range(1, NE + 1):
        m = jnp.max(jnp.where(sp < cur_v, sp, _NEG), axis=1, keepdims=True)
        t = jnp.where(j == e, m, t)
        cur_v = m
    t = jnp.where(j > NE, cur_v, t)

    z_ref[...] = jnp.where(sp >= t, jnp.maximum(sp, 0.0), 0.0)


def kernel(x, W):
    B, D = x.shape
    N = W.shape[0]
    nb = B // RB
    wt = W.T
    return pl.pallas_call(
        _body,
        grid=(nb + 1,),
        in_specs=[
            pl.BlockSpec((RB, D), lambda i: (jnp.minimum(i, nb - 1), 0)),
            pl.BlockSpec((D, N), lambda i: (0, 0)),
        ],
        out_specs=pl.BlockSpec((RB, N), lambda i: (jnp.maximum(i - 1, 0), 0)),
        out_shape=jax.ShapeDtypeStruct((B, N), jnp.float32),
        scratch_shapes=[pltpu.VMEM((2, RB, N), jnp.float32)],
    )(x, wt)


# speedup vs baseline: 1.4920x; 1.4920x over previous
import jax
import jax.numpy as jnp
from jax.experimental import pallas as pl
from jax.experimental.pallas import tpu as pltpu

K = 64
RB = 128
NF = 10
NE = 7
_NEG = -3.0e38


def _body(x_ref, wt_ref, z_ref, s_ref):
    x = x_ref[...]
    xn = x * jax.lax.rsqrt(jnp.maximum(jnp.sum(x * x, axis=1, keepdims=True),
                                       1e-24))
    s = jnp.dot(xn, wt_ref[...], preferred_element_type=jnp.float32)
    s_ref[...] = s

    hi = jnp.max(s, axis=1, keepdims=True) + 1e-6
    lo = jnp.zeros_like(hi)
    chi = jnp.zeros_like(hi)

    def fstep(_, carry):
        lo, hi, chi = carry
        mid = (lo + hi) * 0.5
        cnt = jnp.sum((s_ref[...] >= mid).astype(jnp.float32), axis=1,
                      keepdims=True)
        ge = cnt >= K
        return (jnp.where(ge, mid, lo), jnp.where(ge, hi, mid),
                jnp.where(ge, chi, cnt))

    lo, hi, chi = jax.lax.fori_loop(0, NF, fstep, (lo, hi, chi))

    s = s_ref[...]
    j = K - chi

    cur = hi
    t = jnp.full_like(hi, _NEG)
    for e in range(1, NE + 1):
        m = jnp.max(jnp.where(s < cur, s, _NEG), axis=1, keepdims=True)
        t = jnp.where(j == e, m, t)
        cur = m
    t = jnp.where(j > NE, cur, t)

    z_ref[...] = jnp.where(s >= t, jnp.maximum(s, 0.0), 0.0)


def kernel(x, W):
    B, D = x.shape
    N = W.shape[0]
    wt = W.T
    return pl.pallas_call(
        _body,
        grid=(B // RB,),
        in_specs=[
            pl.BlockSpec((RB, D), lambda i: (i, 0)),
            pl.BlockSpec((D, N), lambda i: (0, 0)),
        ],
        out_specs=pl.BlockSpec((RB, N), lambda i: (i, 0)),
        out_shape=jax.ShapeDtypeStruct((B, N), jnp.float32),
        scratch_shapes=[pltpu.VMEM((RB, N), jnp.float32)],
    )(x, wt)


# speedup vs baseline: 1.5326x; 1.0272x over previous
import jax
import jax.numpy as jnp
from jax.experimental import pallas as pl
from jax.experimental.pallas import tpu as pltpu

K = 64
RB = 128
NF = 11
NE = 5
_NEG = -3.0e38


def _body(x_ref, wt_ref, z_ref, s_ref):
    x = x_ref[...]
    xn = x * jax.lax.rsqrt(jnp.maximum(jnp.sum(x * x, axis=1, keepdims=True),
                                       1e-24))
    s = jnp.dot(xn, wt_ref[...], preferred_element_type=jnp.float32)
    s_ref[...] = s

    hi = jnp.max(s, axis=1, keepdims=True) + 1e-6
    lo = jnp.zeros_like(hi)
    chi = jnp.zeros_like(hi)

    def fstep(_, carry):
        lo, hi, chi = carry
        mid = (lo + hi) * 0.5
        cnt = jnp.sum((s_ref[...] >= mid).astype(jnp.float32), axis=1,
                      keepdims=True)
        ge = cnt >= K
        return (jnp.where(ge, mid, lo), jnp.where(ge, hi, mid),
                jnp.where(ge, chi, cnt))

    lo, hi, chi = jax.lax.fori_loop(0, NF, fstep, (lo, hi, chi))

    s = s_ref[...]
    j = K - chi

    cur = hi
    t = jnp.full_like(hi, _NEG)
    for e in range(1, NE + 1):
        m = jnp.max(jnp.where(s < cur, s, _NEG), axis=1, keepdims=True)
        t = jnp.where(j == e, m, t)
        cur = m
    t = jnp.where(j > NE, cur, t)

    z_ref[...] = jnp.where(s >= t, jnp.maximum(s, 0.0), 0.0)


def kernel(x, W):
    B, D = x.shape
    N = W.shape[0]
    wt = W.T
    return pl.pallas_call(
        _body,
        grid=(B // RB,),
        in_specs=[
            pl.BlockSpec((RB, D), lambda i: (i, 0)),
            pl.BlockSpec((D, N), lambda i: (0, 0)),
        ],
        out_specs=pl.BlockSpec((RB, N), lambda i: (i, 0)),
        out_shape=jax.ShapeDtypeStruct((B, N), jnp.float32),
        scratch_shapes=[pltpu.VMEM((RB, N), jnp.float32)],
    )(x, wt)
